# Initial kernel scaffold; baseline (speedup 1.0000x reference)
import functools

import jax
import jax.numpy as jnp
from jax import lax
from jax.experimental import pallas as pl
from jax.experimental.pallas import tpu as pltpu
from jax.experimental.pallas import tpu_sc as plsc

N = 10000
E = 320000
NC = 2
NS = 16
NW = NC * NS
L = 16
K = 128
NCH = 80
EPAD = NW * NCH * K
ACC_ROWS = 10240
ZROWS = 64


def _make_agg(D):
    mesh = plsc.VectorSubcoreMesh(core_axis_name="c", subcore_axis_name="s")

    @functools.partial(
        pl.kernel,
        mesh=mesh,
        out_type=jax.ShapeDtypeStruct((NC, N, D), jnp.float32),
        scratch_types=[
            pltpu.VMEM_SHARED((ACC_ROWS, D), jnp.float32),
            pltpu.VMEM((NCH, K), jnp.int32),
            pltpu.VMEM((NCH, K), jnp.int32),
            pltpu.VMEM((K, D), jnp.float32),
            pltpu.VMEM((K, D), jnp.float32),
            pltpu.VMEM((ZROWS, D), jnp.float32),
            pltpu.SemaphoreType.DMA,
            pltpu.SemaphoreType.DMA,
        ],
    )
    def agg(h_hbm, src_hbm, dst_hbm, out_hbm, acc, sidx, didx, bufa, bufb,
            zbuf, sema, semb):
        c = lax.axis_index("c")
        s = lax.axis_index("s")
        wid = c * NS + s

        pltpu.sync_copy(src_hbm.at[wid], sidx)
        pltpu.sync_copy(dst_hbm.at[wid], didx)

        @pl.loop(0, ZROWS)
        def _(r):
            @pl.loop(0, D, step=L)
            def _(col):
                zbuf[r, pl.ds(col, L)] = jnp.zeros((L,), jnp.float32)

        rows_per_tile = ACC_ROWS // NS
        @pl.loop(0, rows_per_tile, step=ZROWS)
        def _(r):
            pltpu.sync_copy(zbuf, acc.at[pl.ds(s * rows_per_tile + r, ZROWS)])

        plsc.subcore_barrier()

        @pl.loop(0, NCH, step=2)
        def _(j):
            cpa = pltpu.async_copy(h_hbm.at[sidx.at[j]], bufa, sema)
            cpb = pltpu.async_copy(h_hbm.at[sidx.at[j + 1]], bufb, semb)
            cpa.wait()
            pltpu.sync_copy(bufa, acc.at[didx.at[j]], add=True)
            cpb.wait()
            pltpu.sync_copy(bufb, acc.at[didx.at[j + 1]], add=True)

        plsc.subcore_barrier()

        out_rows = N // NS
        pltpu.sync_copy(acc.at[pl.ds(s * out_rows, out_rows)],
                        out_hbm.at[c, pl.ds(s * out_rows, out_rows)])

    return agg


_agg128 = _make_agg(128)
_agg64 = _make_agg(64)

_BM = 1000


def _mm_body(x_ref, w_ref, o_ref):
    o_ref[...] = jnp.dot(x_ref[...], w_ref[...],
                         preferred_element_type=jnp.float32,
                         precision=lax.Precision.HIGHEST)


def _matmul(x, W):
    M, Kd = x.shape
    D = W.shape[1]
    return pl.pallas_call(
        _mm_body,
        grid=(M // _BM,),
        in_specs=[pl.BlockSpec((_BM, Kd), lambda i: (i, 0)),
                  pl.BlockSpec((Kd, D), lambda i: (0, 0))],
        out_specs=pl.BlockSpec((_BM, D), lambda i: (i, 0)),
        out_shape=jax.ShapeDtypeStruct((M, D), jnp.float32),
    )(x, W)


def _fused_body(p_ref, b_ref, w_ref, o_ref):
    a = jax.nn.relu(p_ref[0] + p_ref[1] + b_ref[...])
    o_ref[...] = jnp.dot(a, w_ref[...],
                         preferred_element_type=jnp.float32,
                         precision=lax.Precision.HIGHEST)


def _fused_matmul(p, b, W):
    _, M, Kd = p.shape
    D = W.shape[1]
    return pl.pallas_call(
        _fused_body,
        grid=(M // _BM,),
        in_specs=[pl.BlockSpec((2, _BM, Kd), lambda i: (0, i, 0)),
                  pl.BlockSpec((1, Kd), lambda i: (0, 0)),
                  pl.BlockSpec((Kd, D), lambda i: (0, 0))],
        out_specs=pl.BlockSpec((_BM, D), lambda i: (i, 0)),
        out_shape=jax.ShapeDtypeStruct((M, D), jnp.float32),
    )(p, b.reshape(1, Kd), W)


def _combine_body(p_ref, b_ref, o_ref):
    o_ref[...] = p_ref[0] + p_ref[1] + b_ref[...]


def _combine(p, b):
    _, M, D = p.shape
    return pl.pallas_call(
        _combine_body,
        grid=(M // _BM,),
        in_specs=[pl.BlockSpec((2, _BM, D), lambda i: (0, i, 0)),
                  pl.BlockSpec((1, D), lambda i: (0, 0))],
        out_specs=pl.BlockSpec((_BM, D), lambda i: (i, 0)),
        out_shape=jax.ShapeDtypeStruct((M, D), jnp.float32),
    )(p, b.reshape(1, D))


def kernel(x, edge_index, W0, b0, W1, b1, W2, b2):
    src = edge_index[0].astype(jnp.int32)
    dst = edge_index[1].astype(jnp.int32)
    pad = EPAD - E
    pad_idx = jnp.arange(pad, dtype=jnp.int32)
    src3 = jnp.concatenate([src, pad_idx % N]).reshape(NW, NCH, K)
    dst3 = jnp.concatenate([dst, N + pad_idx % (ACC_ROWS - N)]).reshape(NW, NCH, K)

    h0 = _matmul(x, W0)
    p0 = _agg128(h0, src3, dst3)
    h1 = _fused_matmul(p0, b0, W1)
    p1 = _agg128(h1, src3, dst3)
    h2 = _fused_matmul(p1, b1, W2)
    p2 = _agg64(h2, src3, dst3)
    return _combine(p2, b2)


# speedup vs baseline: 8.6596x; 8.6596x over previous
import functools

import jax
import jax.numpy as jnp
from jax import lax
from jax.experimental import pallas as pl
from jax.experimental.pallas import tpu as pltpu
from jax.experimental.pallas import tpu_sc as plsc

N = 10000
E = 320000
NC = 2
NS = 16
NW = NC * NS
L = 16
K = 128
NCH = 80
EPAD = NW * NCH * K
ACC_ROWS = 10240


def _make_agg(D):
    mesh = plsc.VectorSubcoreMesh(core_axis_name="c", subcore_axis_name="s")

    @functools.partial(
        pl.kernel,
        mesh=mesh,
        compiler_params=pltpu.CompilerParams(use_tc_tiling_on_sc=False),
        out_type=jax.ShapeDtypeStruct((NC, N, D), jnp.float32),
        scratch_types=[
            pltpu.VMEM_SHARED((ACC_ROWS, D), jnp.float32),
            pltpu.VMEM((NCH, K), jnp.int32),
            pltpu.VMEM((NCH, K), jnp.int32),
            pltpu.VMEM((K, D), jnp.float32),
            pltpu.SemaphoreType.DMA,
        ],
    )
    def agg(h_hbm, src_hbm, dst_hbm, out_hbm, acc, sidx, didx, bufa, sema):
        c = lax.axis_index("c")
        s = lax.axis_index("s")
        wid = c * NS + s

        pltpu.sync_copy(src_hbm.at[wid], sidx)
        pltpu.sync_copy(dst_hbm.at[wid], didx)

        @pl.loop(0, K)
        def _(r):
            @pl.loop(0, D, step=L)
            def _(col):
                bufa[r, pl.ds(col, L)] = jnp.zeros((L,), jnp.float32)

        rows_per_tile = ACC_ROWS // NS
        @pl.loop(0, rows_per_tile, step=K)
        def _(r):
            pltpu.sync_copy(bufa, acc.at[pl.ds(s * rows_per_tile + r, K)])

        plsc.subcore_barrier()

        @pl.loop(0, NCH)
        def _(j):
            pltpu.async_copy(h_hbm.at[sidx.at[j]], bufa, sema).wait()
            pltpu.sync_copy(bufa, acc.at[didx.at[j]], add=True)

        plsc.subcore_barrier()

        out_rows = 624
        pltpu.sync_copy(acc.at[pl.ds(s * out_rows, out_rows)],
                        out_hbm.at[c, pl.ds(s * out_rows, out_rows)])

        @pl.when(s == 0)
        def _():
            pltpu.sync_copy(acc.at[pl.ds(NS * out_rows, N - NS * out_rows)],
                            out_hbm.at[c, pl.ds(NS * out_rows, N - NS * out_rows)])

    return agg


_agg128 = _make_agg(128)
_agg64 = _make_agg(64)

_BM = 1000


def _mm_body(x_ref, w_ref, o_ref):
    o_ref[...] = jnp.dot(x_ref[...], w_ref[...],
                         preferred_element_type=jnp.float32,
                         precision=lax.Precision.HIGHEST)


def _matmul(x, W):
    M, Kd = x.shape
    D = W.shape[1]
    return pl.pallas_call(
        _mm_body,
        grid=(M // _BM,),
        in_specs=[pl.BlockSpec((_BM, Kd), lambda i: (i, 0)),
                  pl.BlockSpec((Kd, D), lambda i: (0, 0))],
        out_specs=pl.BlockSpec((_BM, D), lambda i: (i, 0)),
        out_shape=jax.ShapeDtypeStruct((M, D), jnp.float32),
    )(x, W)


def _fused_body(p_ref, b_ref, w_ref, o_ref):
    a = jax.nn.relu(p_ref[0] + p_ref[1] + b_ref[...])
    o_ref[...] = jnp.dot(a, w_ref[...],
                         preferred_element_type=jnp.float32,
                         precision=lax.Precision.HIGHEST)


def _fused_matmul(p, b, W):
    _, M, Kd = p.shape
    D = W.shape[1]
    return pl.pallas_call(
        _fused_body,
        grid=(M // _BM,),
        in_specs=[pl.BlockSpec((2, _BM, Kd), lambda i: (0, i, 0)),
                  pl.BlockSpec((1, Kd), lambda i: (0, 0)),
                  pl.BlockSpec((Kd, D), lambda i: (0, 0))],
        out_specs=pl.BlockSpec((_BM, D), lambda i: (i, 0)),
        out_shape=jax.ShapeDtypeStruct((M, D), jnp.float32),
    )(p, b.reshape(1, Kd), W)


def _combine_body(p_ref, b_ref, o_ref):
    o_ref[...] = p_ref[0] + p_ref[1] + b_ref[...]


def _combine(p, b):
    _, M, D = p.shape
    return pl.pallas_call(
        _combine_body,
        grid=(M // _BM,),
        in_specs=[pl.BlockSpec((2, _BM, D), lambda i: (0, i, 0)),
                  pl.BlockSpec((1, D), lambda i: (0, 0))],
        out_specs=pl.BlockSpec((_BM, D), lambda i: (i, 0)),
        out_shape=jax.ShapeDtypeStruct((M, D), jnp.float32),
    )(p, b.reshape(1, D))


def kernel(x, edge_index, W0, b0, W1, b1, W2, b2):
    src = edge_index[0].astype(jnp.int32)
    dst = edge_index[1].astype(jnp.int32)
    pad = EPAD - E
    pad_idx = jnp.arange(pad, dtype=jnp.int32)
    src3 = jnp.concatenate([src, pad_idx % N]).reshape(NW, NCH, K)
    dst3 = jnp.concatenate([dst, N + pad_idx % (ACC_ROWS - N)]).reshape(NW, NCH, K)

    h0 = _matmul(x, W0)
    p0 = _agg128(h0, src3, dst3)
    h1 = _fused_matmul(p0, b0, W1)
    p1 = _agg128(h1, src3, dst3)
    h2 = _fused_matmul(p1, b1, W2)
    p2 = _agg64(h2, src3, dst3)
    return _combine(p2, b2)


# speedup vs baseline: 12.7403x; 1.4712x over previous
import functools

import jax
import jax.numpy as jnp
from jax import lax
from jax.experimental import pallas as pl
from jax.experimental.pallas import tpu as pltpu
from jax.experimental.pallas import tpu_sc as plsc

N = 10000
E = 320000
NC = 2
NS = 16
NW = NC * NS
L = 16
K = 128
NCH = 80
HNCH = NCH // 2
EPAD = NW * NCH * K
ACC_ROWS = 10240


def _make_agg(D):
    mesh = plsc.VectorSubcoreMesh(core_axis_name="c", subcore_axis_name="s")

    @functools.partial(
        pl.kernel,
        mesh=mesh,
        compiler_params=pltpu.CompilerParams(use_tc_tiling_on_sc=False),
        out_type=jax.ShapeDtypeStruct((NC, N, D), jnp.float32),
        scratch_types=[
            pltpu.VMEM_SHARED((ACC_ROWS, D), jnp.float32),
            pltpu.VMEM((HNCH, K), jnp.int32),
            pltpu.VMEM((HNCH, K), jnp.int32),
            pltpu.VMEM((K, D), jnp.float32),
            pltpu.VMEM((K, D), jnp.float32),
            pltpu.SemaphoreType.DMA,
            pltpu.SemaphoreType.DMA,
        ],
    )
    def agg(h_hbm, src_hbm, dst_hbm, out_hbm, acc, sidx, didx, bufa, bufb,
            sema, semb):
        c = lax.axis_index("c")
        s = lax.axis_index("s")
        wid = c * NS + s

        @pl.loop(0, K)
        def _(r):
            @pl.loop(0, D, step=L)
            def _(col):
                bufa[r, pl.ds(col, L)] = jnp.zeros((L,), jnp.float32)

        rows_per_tile = ACC_ROWS // NS
        @pl.loop(0, rows_per_tile, step=K)
        def _(r):
            pltpu.sync_copy(bufa, acc.at[pl.ds(s * rows_per_tile + r, K)])

        plsc.subcore_barrier()

        def _drain(sem):
            pltpu.make_async_copy(h_hbm.at[pl.ds(0, K)], bufa, sem).wait()

        for half in range(2):
            pltpu.sync_copy(src_hbm.at[wid, pl.ds(half * HNCH, HNCH)], sidx)
            pltpu.sync_copy(dst_hbm.at[wid, pl.ds(half * HNCH, HNCH)], didx)
            pltpu.async_copy(h_hbm.at[sidx.at[0]], bufa, sema)
            pltpu.async_copy(h_hbm.at[sidx.at[1]], bufb, semb)

            @pl.loop(0, HNCH - 2, step=2)
            def _(j):
                _drain(sema)
                pltpu.sync_copy(bufa, acc.at[didx.at[j]], add=True)
                pltpu.async_copy(h_hbm.at[sidx.at[j + 2]], bufa, sema)
                _drain(semb)
                pltpu.sync_copy(bufb, acc.at[didx.at[j + 1]], add=True)
                pltpu.async_copy(h_hbm.at[sidx.at[j + 3]], bufb, semb)

            _drain(sema)
            pltpu.sync_copy(bufa, acc.at[didx.at[HNCH - 2]], add=True)
            _drain(semb)
            pltpu.sync_copy(bufb, acc.at[didx.at[HNCH - 1]], add=True)

        plsc.subcore_barrier()

        out_rows = 624
        pltpu.sync_copy(acc.at[pl.ds(s * out_rows, out_rows)],
                        out_hbm.at[c, pl.ds(s * out_rows, out_rows)])

        @pl.when(s == 0)
        def _():
            pltpu.sync_copy(acc.at[pl.ds(NS * out_rows, N - NS * out_rows)],
                            out_hbm.at[c, pl.ds(NS * out_rows, N - NS * out_rows)])

    return agg


_agg128 = _make_agg(128)
_agg64 = _make_agg(64)

_BM = 1000


def _mm_body(x_ref, w_ref, o_ref):
    o_ref[...] = jnp.dot(x_ref[...], w_ref[...],
                         preferred_element_type=jnp.float32,
                         precision=lax.Precision.HIGHEST)


def _matmul(x, W):
    M, Kd = x.shape
    D = W.shape[1]
    return pl.pallas_call(
        _mm_body,
        grid=(M // _BM,),
        in_specs=[pl.BlockSpec((_BM, Kd), lambda i: (i, 0)),
                  pl.BlockSpec((Kd, D), lambda i: (0, 0))],
        out_specs=pl.BlockSpec((_BM, D), lambda i: (i, 0)),
        out_shape=jax.ShapeDtypeStruct((M, D), jnp.float32),
    )(x, W)


def _fused_body(p_ref, b_ref, w_ref, o_ref):
    a = jax.nn.relu(p_ref[0] + p_ref[1] + b_ref[...])
    o_ref[...] = jnp.dot(a, w_ref[...],
                         preferred_element_type=jnp.float32,
                         precision=lax.Precision.HIGHEST)


def _fused_matmul(p, b, W):
    _, M, Kd = p.shape
    D = W.shape[1]
    return pl.pallas_call(
        _fused_body,
        grid=(M // _BM,),
        in_specs=[pl.BlockSpec((2, _BM, Kd), lambda i: (0, i, 0)),
                  pl.BlockSpec((1, Kd), lambda i: (0, 0)),
                  pl.BlockSpec((Kd, D), lambda i: (0, 0))],
        out_specs=pl.BlockSpec((_BM, D), lambda i: (i, 0)),
        out_shape=jax.ShapeDtypeStruct((M, D), jnp.float32),
    )(p, b.reshape(1, Kd), W)


def _combine_body(p_ref, b_ref, o_ref):
    o_ref[...] = p_ref[0] + p_ref[1] + b_ref[...]


def _combine(p, b):
    _, M, D = p.shape
    return pl.pallas_call(
        _combine_body,
        grid=(M // _BM,),
        in_specs=[pl.BlockSpec((2, _BM, D), lambda i: (0, i, 0)),
                  pl.BlockSpec((1, D), lambda i: (0, 0))],
        out_specs=pl.BlockSpec((_BM, D), lambda i: (i, 0)),
        out_shape=jax.ShapeDtypeStruct((M, D), jnp.float32),
    )(p, b.reshape(1, D))


def kernel(x, edge_index, W0, b0, W1, b1, W2, b2):
    src = edge_index[0].astype(jnp.int32)
    dst = edge_index[1].astype(jnp.int32)
    pad = EPAD - E
    pad_idx = jnp.arange(pad, dtype=jnp.int32)
    src3 = jnp.concatenate([src, pad_idx % N]).reshape(NW, NCH, K)
    dst3 = jnp.concatenate([dst, N + pad_idx % (ACC_ROWS - N)]).reshape(NW, NCH, K)

    h0 = _matmul(x, W0)
    p0 = _agg128(h0, src3, dst3)
    h1 = _fused_matmul(p0, b0, W1)
    p1 = _agg128(h1, src3, dst3)
    h2 = _fused_matmul(p1, b1, W2)
    p2 = _agg64(h2, src3, dst3)
    return _combine(p2, b2)


# speedup vs baseline: 13.0742x; 1.0262x over previous
import functools

import jax
import jax.numpy as jnp
from jax import lax
from jax.experimental import pallas as pl
from jax.experimental.pallas import tpu as pltpu
from jax.experimental.pallas import tpu_sc as plsc

N = 10000
E = 320000
NC = 2
NS = 16
NW = NC * NS
L = 16
K = 128
NCH = 80
HNCH = NCH // 2
EPAD = NW * NCH * K
ACC_ROWS = 10240


def _make_agg(D):
    mesh = plsc.VectorSubcoreMesh(core_axis_name="c", subcore_axis_name="s")

    @functools.partial(
        pl.kernel,
        mesh=mesh,
        compiler_params=pltpu.CompilerParams(use_tc_tiling_on_sc=False),
        out_type=jax.ShapeDtypeStruct((NC, N, D), jnp.float32),
        scratch_types=[
            pltpu.VMEM_SHARED((ACC_ROWS, D), jnp.float32),
            pltpu.VMEM((HNCH, K), jnp.int32),
            pltpu.VMEM((HNCH, K), jnp.int32),
            pltpu.VMEM((K, D), jnp.float32),
            pltpu.VMEM((K, D), jnp.float32),
            pltpu.SemaphoreType.DMA,
            pltpu.SemaphoreType.DMA,
        ],
    )
    def agg(h_hbm, src_hbm, dst_hbm, out_hbm, acc, sidx, didx, bufa, bufb,
            sema, semb):
        c = lax.axis_index("c")
        s = lax.axis_index("s")
        wid = c * NS + s

        @pl.loop(0, K)
        def _(r):
            @pl.loop(0, D, step=L)
            def _(col):
                bufa[r, pl.ds(col, L)] = jnp.zeros((L,), jnp.float32)

        rows_per_tile = ACC_ROWS // NS
        @pl.loop(0, rows_per_tile, step=K)
        def _(r):
            pltpu.sync_copy(bufa, acc.at[pl.ds(s * rows_per_tile + r, K)])

        plsc.subcore_barrier()

        def _drain(sem):
            pltpu.make_async_copy(h_hbm.at[pl.ds(0, K)], bufa, sem).wait()

        for half in range(2):
            pltpu.sync_copy(src_hbm.at[wid, pl.ds(half * HNCH, HNCH)], sidx)
            pltpu.sync_copy(dst_hbm.at[wid, pl.ds(half * HNCH, HNCH)], didx)
            pltpu.async_copy(h_hbm.at[sidx.at[0]], bufa, sema)
            pltpu.async_copy(h_hbm.at[sidx.at[1]], bufb, semb)

            @pl.loop(0, HNCH - 2, step=2)
            def _(j):
                _drain(sema)
                pltpu.sync_copy(bufa, acc.at[didx.at[j]], add=True)
                pltpu.async_copy(h_hbm.at[sidx.at[j + 2]], bufa, sema)
                _drain(semb)
                pltpu.sync_copy(bufb, acc.at[didx.at[j + 1]], add=True)
                pltpu.async_copy(h_hbm.at[sidx.at[j + 3]], bufb, semb)

            _drain(sema)
            pltpu.sync_copy(bufa, acc.at[didx.at[HNCH - 2]], add=True)
            _drain(semb)
            pltpu.sync_copy(bufb, acc.at[didx.at[HNCH - 1]], add=True)

        plsc.subcore_barrier()

        out_rows = 624
        pltpu.sync_copy(acc.at[pl.ds(s * out_rows, out_rows)],
                        out_hbm.at[c, pl.ds(s * out_rows, out_rows)])

        @pl.when(s == 0)
        def _():
            pltpu.sync_copy(acc.at[pl.ds(NS * out_rows, N - NS * out_rows)],
                            out_hbm.at[c, pl.ds(NS * out_rows, N - NS * out_rows)])

    return agg


_agg128 = _make_agg(128)
_agg64 = _make_agg(64)

_BM = 1000


def _mm_body(x_ref, w_ref, o_ref):
    o_ref[...] = jnp.dot(x_ref[...], w_ref[...],
                         preferred_element_type=jnp.float32)


def _matmul(x, W):
    M, Kd = x.shape
    D = W.shape[1]
    return pl.pallas_call(
        _mm_body,
        grid=(M // _BM,),
        in_specs=[pl.BlockSpec((_BM, Kd), lambda i: (i, 0)),
                  pl.BlockSpec((Kd, D), lambda i: (0, 0))],
        out_specs=pl.BlockSpec((_BM, D), lambda i: (i, 0)),
        out_shape=jax.ShapeDtypeStruct((M, D), jnp.float32),
    )(x, W)


def _fused_body(p_ref, b_ref, w_ref, o_ref):
    a = jax.nn.relu(p_ref[0] + p_ref[1] + b_ref[...])
    o_ref[...] = jnp.dot(a, w_ref[...],
                         preferred_element_type=jnp.float32)


def _fused_matmul(p, b, W):
    _, M, Kd = p.shape
    D = W.shape[1]
    return pl.pallas_call(
        _fused_body,
        grid=(M // _BM,),
        in_specs=[pl.BlockSpec((2, _BM, Kd), lambda i: (0, i, 0)),
                  pl.BlockSpec((1, Kd), lambda i: (0, 0)),
                  pl.BlockSpec((Kd, D), lambda i: (0, 0))],
        out_specs=pl.BlockSpec((_BM, D), lambda i: (i, 0)),
        out_shape=jax.ShapeDtypeStruct((M, D), jnp.float32),
    )(p, b.reshape(1, Kd), W)


def _combine_body(p_ref, b_ref, o_ref):
    o_ref[...] = p_ref[0] + p_ref[1] + b_ref[...]


def _combine(p, b):
    _, M, D = p.shape
    return pl.pallas_call(
        _combine_body,
        grid=(M // _BM,),
        in_specs=[pl.BlockSpec((2, _BM, D), lambda i: (0, i, 0)),
                  pl.BlockSpec((1, D), lambda i: (0, 0))],
        out_specs=pl.BlockSpec((_BM, D), lambda i: (i, 0)),
        out_shape=jax.ShapeDtypeStruct((M, D), jnp.float32),
    )(p, b.reshape(1, D))


def kernel(x, edge_index, W0, b0, W1, b1, W2, b2):
    src = edge_index[0].astype(jnp.int32)
    dst = edge_index[1].astype(jnp.int32)
    pad = EPAD - E
    pad_idx = jnp.arange(pad, dtype=jnp.int32)
    src3 = jnp.concatenate([src, pad_idx % N]).reshape(NW, NCH, K)
    dst3 = jnp.concatenate([dst, N + pad_idx % (ACC_ROWS - N)]).reshape(NW, NCH, K)

    h0 = _matmul(x, W0)
    p0 = _agg128(h0, src3, dst3)
    h1 = _fused_matmul(p0, b0, W1)
    p1 = _agg128(h1, src3, dst3)
    h2 = _fused_matmul(p1, b1, W2)
    p2 = _agg64(h2, src3, dst3)
    return _combine(p2, b2)


# speedup vs baseline: 14.0868x; 1.0775x over previous
import functools

import jax
import jax.numpy as jnp
from jax import lax
from jax.experimental import pallas as pl
from jax.experimental.pallas import tpu as pltpu
from jax.experimental.pallas import tpu_sc as plsc

N = 10000
E = 320000
NC = 2
NS = 16
NW = NC * NS
L = 16
K = 128
NCH = 80
HNCH = NCH // 2
EPAD = NW * NCH * K
ACC_ROWS = 10240


def _make_agg(D, depth, nblk):
    mesh = plsc.VectorSubcoreMesh(core_axis_name="c", subcore_axis_name="s")
    BNCH = NCH // nblk

    @functools.partial(
        pl.kernel,
        mesh=mesh,
        compiler_params=pltpu.CompilerParams(use_tc_tiling_on_sc=False),
        out_type=jax.ShapeDtypeStruct((NC, N, D), jnp.float32),
        scratch_types=[
            pltpu.VMEM_SHARED((ACC_ROWS, D), jnp.float32),
            pltpu.VMEM((BNCH, K), jnp.int32),
            pltpu.VMEM((BNCH, K), jnp.int32),
        ] + [pltpu.VMEM((K, D), jnp.float32) for _ in range(depth)]
          + [pltpu.SemaphoreType.DMA for _ in range(depth)],
    )
    def agg(h_hbm, src_hbm, dst_hbm, out_hbm, acc, sidx, didx, *bufs_sems):
        bufs = bufs_sems[:depth]
        sems = bufs_sems[depth:]
        c = lax.axis_index("c")
        s = lax.axis_index("s")
        wid = c * NS + s

        @pl.loop(0, K)
        def _(r):
            @pl.loop(0, D, step=L)
            def _(col):
                bufs[0][r, pl.ds(col, L)] = jnp.zeros((L,), jnp.float32)

        rows_per_tile = ACC_ROWS // NS
        @pl.loop(0, rows_per_tile, step=K)
        def _(r):
            pltpu.sync_copy(bufs[0], acc.at[pl.ds(s * rows_per_tile + r, K)])

        def _drain(q):
            pltpu.make_async_copy(h_hbm.at[pl.ds(0, K)], bufs[q], sems[q]).wait()

        pltpu.sync_copy(src_hbm.at[wid, pl.ds(0, BNCH)], sidx)
        pltpu.sync_copy(dst_hbm.at[wid, pl.ds(0, BNCH)], didx)
        for q in range(depth):
            pltpu.async_copy(h_hbm.at[sidx.at[q]], bufs[q], sems[q])

        plsc.subcore_barrier()

        for blk in range(nblk):
            if blk > 0:
                pltpu.sync_copy(src_hbm.at[wid, pl.ds(blk * BNCH, BNCH)], sidx)
                pltpu.sync_copy(dst_hbm.at[wid, pl.ds(blk * BNCH, BNCH)], didx)
                for q in range(depth):
                    pltpu.async_copy(h_hbm.at[sidx.at[q]], bufs[q], sems[q])

            @pl.loop(0, BNCH - depth, step=depth)
            def _(j):
                for q in range(depth):
                    _drain(q)
                    pltpu.sync_copy(bufs[q], acc.at[didx.at[j + q]], add=True)
                    pltpu.async_copy(h_hbm.at[sidx.at[j + depth + q]],
                                     bufs[q], sems[q])

            for q in range(depth):
                _drain(q)
                pltpu.sync_copy(bufs[q], acc.at[didx.at[BNCH - depth + q]],
                                add=True)

        plsc.subcore_barrier()

        out_rows = 624
        pltpu.sync_copy(acc.at[pl.ds(s * out_rows, out_rows)],
                        out_hbm.at[c, pl.ds(s * out_rows, out_rows)])

        @pl.when(s == 0)
        def _():
            pltpu.sync_copy(acc.at[pl.ds(NS * out_rows, N - NS * out_rows)],
                            out_hbm.at[c, pl.ds(NS * out_rows, N - NS * out_rows)])

    return agg


_agg128 = _make_agg(128, depth=2, nblk=2)
_agg64 = _make_agg(64, depth=4, nblk=1)

_BM = 2000


def _mm_body(x_ref, w_ref, o_ref):
    o_ref[...] = jnp.dot(x_ref[...], w_ref[...],
                         preferred_element_type=jnp.float32)


def _matmul(x, W):
    M, Kd = x.shape
    D = W.shape[1]
    return pl.pallas_call(
        _mm_body,
        grid=(M // _BM,),
        in_specs=[pl.BlockSpec((_BM, Kd), lambda i: (i, 0)),
                  pl.BlockSpec((Kd, D), lambda i: (0, 0))],
        out_specs=pl.BlockSpec((_BM, D), lambda i: (i, 0)),
        out_shape=jax.ShapeDtypeStruct((M, D), jnp.float32),
    )(x, W)


def _fused_body(p_ref, b_ref, w_ref, o_ref):
    a = jax.nn.relu(p_ref[0] + p_ref[1] + b_ref[...])
    o_ref[...] = jnp.dot(a, w_ref[...],
                         preferred_element_type=jnp.float32)


def _fused_matmul(p, b, W):
    _, M, Kd = p.shape
    D = W.shape[1]
    return pl.pallas_call(
        _fused_body,
        grid=(M // _BM,),
        in_specs=[pl.BlockSpec((2, _BM, Kd), lambda i: (0, i, 0)),
                  pl.BlockSpec((1, Kd), lambda i: (0, 0)),
                  pl.BlockSpec((Kd, D), lambda i: (0, 0))],
        out_specs=pl.BlockSpec((_BM, D), lambda i: (i, 0)),
        out_shape=jax.ShapeDtypeStruct((M, D), jnp.float32),
    )(p, b.reshape(1, Kd), W)


def _combine_body(p_ref, b_ref, o_ref):
    o_ref[...] = p_ref[0] + p_ref[1] + b_ref[...]


def _combine(p, b):
    _, M, D = p.shape
    return pl.pallas_call(
        _combine_body,
        grid=(M // _BM,),
        in_specs=[pl.BlockSpec((2, _BM, D), lambda i: (0, i, 0)),
                  pl.BlockSpec((1, D), lambda i: (0, 0))],
        out_specs=pl.BlockSpec((_BM, D), lambda i: (i, 0)),
        out_shape=jax.ShapeDtypeStruct((M, D), jnp.float32),
    )(p, b.reshape(1, D))


def kernel(x, edge_index, W0, b0, W1, b1, W2, b2):
    src = edge_index[0].astype(jnp.int32)
    dst = edge_index[1].astype(jnp.int32)
    pad = EPAD - E
    pad_idx = jnp.arange(pad, dtype=jnp.int32)
    src3 = jnp.concatenate([src, pad_idx % N]).reshape(NW, NCH, K)
    dst3 = jnp.concatenate([dst, N + pad_idx % (ACC_ROWS - N)]).reshape(NW, NCH, K)

    h0 = _matmul(x, W0)
    p0 = _agg128(h0, src3, dst3)
    h1 = _fused_matmul(p0, b0, W1)
    p1 = _agg128(h1, src3, dst3)
    h2 = _fused_matmul(p1, b1, W2)
    p2 = _agg64(h2, src3, dst3)
    return _combine(p2, b2)
